# Initial kernel scaffold; baseline (speedup 1.0000x reference)
import jax
import jax.numpy as jnp
from jax.experimental import pallas as pl


def kernel(value, edge_index, W1, b1, W2, b2, W3, b3, W4, b4, W5, b5):
    raise NotImplementedError("write your pallas kernel here")


# speedup vs baseline: 13.5034x; 13.5034x over previous
import functools

import jax
import jax.numpy as jnp
from jax import lax
from jax.experimental import pallas as pl
from jax.experimental.pallas import tpu as pltpu
from jax.experimental.pallas import tpu_sc as plsc

N = 100000
E = 3200000
B = 128
NSC = 2
NSUB = 16
NW = NSC * NSUB
RT = 25088
RPW = RT // NW
EP = RT * B
NP = 100096
STRIPE = NP // NSUB
SK = 56
NSK = RPW // SK

_DIMS = [21, 8, 16, 32, 64, 4]
_NB = 1000
_NGRID = N // _NB


def _chunking(f):
    fc = f if f < 16 else 16
    return f // fc, fc


def _sc_mesh():
    return plsc.VectorSubcoreMesh(core_axis_name="c", subcore_axis_name="s")


@functools.partial(jax.jit, static_argnums=(0, 1))
def _sc_scatter(nc, fc, hps, srcb, dstb, zeros):
    def body(*refs):
        hp_refs = refs[:nc]
        srcb_r, dstb_r, zeros_r, out_r = refs[nc:nc + 4]
        sidx, didx, vals, agg, sem = refs[nc + 4:]
        cid = lax.axis_index("c")
        sid = lax.axis_index("s")
        wid = cid * NSUB + sid
        r0 = wid * RPW
        st = sid * STRIPE
        for c in range(nc):
            pltpu.sync_copy(zeros_r.at[pl.ds(st, STRIPE)],
                            agg.at[pl.ds(st, STRIPE)])
            plsc.subcore_barrier()

            def super_body(sk, _, c=c):
                row0 = r0 + sk * SK
                pltpu.sync_copy(srcb_r.at[pl.ds(row0, SK)], sidx)
                pltpu.sync_copy(dstb_r.at[pl.ds(row0, SK)], didx)

                def blk_body(j, _):
                    pltpu.async_copy(hp_refs[c].at[sidx.at[j]], vals,
                                     sem).wait()
                    pltpu.sync_copy(vals, agg.at[didx.at[j]], add=True)
                    return 0

                return lax.fori_loop(0, SK, blk_body, 0)

            lax.fori_loop(0, NSK, super_body, 0)
            plsc.subcore_barrier()
            pltpu.sync_copy(agg.at[pl.ds(st, STRIPE)],
                            out_r.at[c, cid, pl.ds(st, STRIPE)])
            plsc.subcore_barrier()

    k = pl.kernel(
        body,
        out_type=jax.ShapeDtypeStruct((nc, NSC, NP, fc), jnp.float32),
        mesh=_sc_mesh(),
        scratch_types=[
            pltpu.VMEM((SK, B), jnp.int32),
            pltpu.VMEM((SK, B), jnp.int32),
            pltpu.VMEM((B, fc), jnp.float32),
            pltpu.VMEM_SHARED((NP, fc), jnp.float32),
            pltpu.SemaphoreType.DMA,
        ],
        compiler_params=pltpu.CompilerParams(use_tc_tiling_on_sc=False),
    )
    return k(*hps, srcb, dstb, zeros)


@jax.jit
def _sc_degree(dstb, ones, zeros):
    def body(dstb_r, ones_r, zeros_r, out_r, didx, ones_v, agg, sem):
        cid = lax.axis_index("c")
        sid = lax.axis_index("s")
        wid = cid * NSUB + sid
        r0 = wid * RPW
        st = sid * STRIPE
        pltpu.sync_copy(ones_r, ones_v)
        pltpu.sync_copy(zeros_r.at[pl.ds(st, STRIPE)],
                        agg.at[pl.ds(st, STRIPE)])
        plsc.subcore_barrier()

        def super_body(sk, _):
            row0 = r0 + sk * SK
            pltpu.sync_copy(dstb_r.at[pl.ds(row0, SK)], didx)

            def blk_body(j, _):
                pltpu.sync_copy(ones_v, agg.at[didx.at[j]], add=True)
                return 0

            return lax.fori_loop(0, SK, blk_body, 0)

        lax.fori_loop(0, NSK, super_body, 0)
        plsc.subcore_barrier()
        pltpu.sync_copy(agg.at[pl.ds(st, STRIPE)],
                        out_r.at[cid, pl.ds(st, STRIPE)])

    k = pl.kernel(
        body,
        out_type=jax.ShapeDtypeStruct((NSC, NP, 8), jnp.float32),
        mesh=_sc_mesh(),
        scratch_types=[
            pltpu.VMEM((SK, B), jnp.int32),
            pltpu.VMEM((B, 8), jnp.float32),
            pltpu.VMEM_SHARED((NP, 8), jnp.float32),
            pltpu.SemaphoreType.DMA,
        ],
        compiler_params=pltpu.CompilerParams(use_tc_tiling_on_sc=False),
    )
    return k(dstb, ones, zeros)


def _dinv_of(deg_ref):
    return lax.rsqrt(deg_ref[0, :, 0:1] + deg_ref[1, :, 0:1] + 1.0)


def _mm_body(nc, fc, x_ref, w_ref, deg_ref, *out_refs):
    dinv = _dinv_of(deg_ref)
    h = jnp.dot(x_ref[...], w_ref[...], preferred_element_type=jnp.float32,
                precision=lax.Precision.HIGHEST) * dinv
    for c in range(nc):
        out_refs[c][...] = h[:, c * fc:(c + 1) * fc]


@functools.partial(jax.jit, static_argnums=(0, 1))
def _tc_matmul(nc, fc, x, w, deg):
    fin = x.shape[1]
    f = w.shape[1]
    return pl.pallas_call(
        functools.partial(_mm_body, nc, fc),
        grid=(_NGRID,),
        in_specs=[
            pl.BlockSpec((_NB, fin), lambda i: (i, 0)),
            pl.BlockSpec((fin, f), lambda i: (0, 0)),
            pl.BlockSpec((NSC, _NB, 8), lambda i: (0, i, 0)),
        ],
        out_specs=[pl.BlockSpec((_NB, fc), lambda i: (i, 0))] * nc,
        out_shape=[jax.ShapeDtypeStruct((N, fc), jnp.float32)] * nc,
    )(x, w, deg)


def _combine_body_multi(nc, agg_ref, hp_ref, deg_ref, b_ref, out_ref):
    dinv = _dinv_of(deg_ref)
    parts = [
        dinv * (agg_ref[c, 0] + agg_ref[c, 1] + hp_ref[c])
        for c in range(nc)
    ]
    t = (parts[0] if nc == 1 else jnp.concatenate(parts, axis=1)) + b_ref[...]
    out_ref[...] = jnp.maximum(t, 0.0)


@functools.partial(jax.jit, static_argnums=(0, 1))
def _tc_combine2(nc, fc, agg, hpc, deg, b):
    f = nc * fc
    return pl.pallas_call(
        functools.partial(_combine_body_multi, nc),
        grid=(_NGRID,),
        in_specs=[
            pl.BlockSpec((nc, NSC, _NB, fc), lambda i: (0, 0, i, 0)),
            pl.BlockSpec((nc, _NB, fc), lambda i: (0, i, 0)),
            pl.BlockSpec((NSC, _NB, 8), lambda i: (0, i, 0)),
            pl.BlockSpec((1, f), lambda i: (0, 0)),
        ],
        out_specs=pl.BlockSpec((_NB, f), lambda i: (i, 0)),
        out_shape=jax.ShapeDtypeStruct((N, f), jnp.float32),
    )(agg, hpc, deg, b.reshape(1, f))


def _final_body(agg_ref, hp_ref, deg_ref, b_ref, out_ref):
    dinv = _dinv_of(deg_ref)
    t = dinv * (agg_ref[0, 0] + agg_ref[0, 1] + hp_ref[...]) + b_ref[...]
    t = jnp.maximum(t, 0.0)
    s = jnp.sum(t, axis=1, keepdims=True)
    out_ref[...] = jax.nn.sigmoid(s)


@jax.jit
def _tc_final(agg, hp, deg, b):
    return pl.pallas_call(
        _final_body,
        grid=(_NGRID,),
        in_specs=[
            pl.BlockSpec((1, NSC, _NB, 8), lambda i: (0, 0, i, 0)),
            pl.BlockSpec((_NB, 8), lambda i: (i, 0)),
            pl.BlockSpec((NSC, _NB, 8), lambda i: (0, i, 0)),
            pl.BlockSpec((1, 8), lambda i: (0, 0)),
        ],
        out_specs=pl.BlockSpec((_NB, 1), lambda i: (i, 0)),
        out_shape=jax.ShapeDtypeStruct((N, 1), jnp.float32),
    )(agg, hp, deg, b.reshape(1, 8))


def kernel(value, edge_index, W1, b1, W2, b2, W3, b3, W4, b4, W5, b5):
    src = edge_index[0]
    dst = edge_index[1]
    npad = EP - E
    srcb = jnp.concatenate([src, jnp.zeros((npad,), jnp.int32)]).reshape(RT, B)
    dstb = jnp.concatenate([dst, jnp.full((npad,), N, jnp.int32)]).reshape(RT, B)

    ones8 = jnp.ones((B, 8), jnp.float32)
    zeros8 = jnp.zeros((NP, 8), jnp.float32)
    deg = _sc_degree(dstb, ones8, zeros8)

    W5p = jnp.concatenate([W5, jnp.zeros((W5.shape[0], 4), jnp.float32)], 1)
    b5p = jnp.concatenate([b5, jnp.zeros((4,), jnp.float32)])
    ws = [W1, W2, W3, W4, W5p]
    bs = [b1, b2, b3, b4, b5p]
    x = value
    for l in range(5):
        f = _DIMS[l + 1] if l < 4 else 8
        nc, fc = _chunking(f)
        hps = _tc_matmul(nc, fc, x, ws[l], deg)
        zeros = jnp.zeros((NP, fc), jnp.float32)
        agg = _sc_scatter(nc, fc, hps, srcb, dstb, zeros)
        if l < 4:
            hpc = jnp.stack(hps, axis=0)
            x = _tc_combine2(nc, fc, agg, hpc, deg, bs[l])
        else:
            out = _tc_final(agg, hps[0], deg, bs[l])
    return out.reshape(N)


# speedup vs baseline: 22.4649x; 1.6636x over previous
import functools

import jax
import jax.numpy as jnp
from jax import lax
from jax.experimental import pallas as pl
from jax.experimental.pallas import tpu as pltpu
from jax.experimental.pallas import tpu_sc as plsc

N = 100000
E = 3200000
B = 128
NSC = 2
NSUB = 16
NW = NSC * NSUB
RT = 25088
RPW = RT // NW
EP = RT * B
NP = 100096
STRIPE = NP // NSUB
SK = 56
NSK = RPW // SK
K = 8
NG = SK // K

_DIMS = [21, 8, 16, 32, 64, 4]
_NB = 1000
_NGRID = N // _NB


def _chunking(f):
    fc = f if f < 16 else 16
    return f // fc, fc


def _sc_mesh():
    return plsc.VectorSubcoreMesh(core_axis_name="c", subcore_axis_name="s")


@functools.partial(jax.jit, static_argnums=(0, 1))
def _sc_scatter(nc, fc, hps, srcb, dstb, zeros):
    def body(*refs):
        hp_refs = refs[:nc]
        srcb_r, dstb_r, zeros_r, out_r = refs[nc:nc + 4]
        sidx, didx, vals, agg, gsem, ssem = refs[nc + 4:]
        cid = lax.axis_index("c")
        sid = lax.axis_index("s")
        wid = cid * NSUB + sid
        r0 = wid * RPW
        st = sid * STRIPE
        for c in range(nc):
            pltpu.sync_copy(zeros_r.at[pl.ds(st, STRIPE)],
                            agg.at[pl.ds(st, STRIPE)])
            plsc.subcore_barrier()

            def super_body(sk, _, c=c):
                row0 = r0 + sk * SK
                pltpu.sync_copy(srcb_r.at[pl.ds(row0, SK)], sidx)
                pltpu.sync_copy(dstb_r.at[pl.ds(row0, SK)], didx)

                def group_body(g, _):
                    j0 = g * K
                    gds = [
                        pltpu.async_copy(hp_refs[c].at[sidx.at[j0 + r]],
                                         vals.at[r], gsem)
                        for r in range(K)
                    ]
                    for gd in gds:
                        gd.wait()
                    sds = [
                        pltpu.async_copy(vals.at[r], agg.at[didx.at[j0 + r]],
                                         ssem, add=True)
                        for r in range(K)
                    ]
                    for sd in sds:
                        sd.wait()
                    return 0

                return lax.fori_loop(0, NG, group_body, 0)

            lax.fori_loop(0, NSK, super_body, 0)
            plsc.subcore_barrier()
            pltpu.sync_copy(agg.at[pl.ds(st, STRIPE)],
                            out_r.at[c, cid, pl.ds(st, STRIPE)])
            plsc.subcore_barrier()

    k = pl.kernel(
        body,
        out_type=jax.ShapeDtypeStruct((nc, NSC, NP, fc), jnp.float32),
        mesh=_sc_mesh(),
        scratch_types=[
            pltpu.VMEM((SK, B), jnp.int32),
            pltpu.VMEM((SK, B), jnp.int32),
            pltpu.VMEM((K, B, fc), jnp.float32),
            pltpu.VMEM_SHARED((NP, fc), jnp.float32),
            pltpu.SemaphoreType.DMA,
            pltpu.SemaphoreType.DMA,
        ],
        compiler_params=pltpu.CompilerParams(use_tc_tiling_on_sc=False),
    )
    return k(*hps, srcb, dstb, zeros)


@jax.jit
def _sc_degree(dstb, ones, zeros):
    def body(dstb_r, ones_r, zeros_r, out_r, didx, ones_v, agg, sem):
        cid = lax.axis_index("c")
        sid = lax.axis_index("s")
        wid = cid * NSUB + sid
        r0 = wid * RPW
        st = sid * STRIPE
        pltpu.sync_copy(ones_r, ones_v)
        pltpu.sync_copy(zeros_r.at[pl.ds(st, STRIPE)],
                        agg.at[pl.ds(st, STRIPE)])
        plsc.subcore_barrier()

        def super_body(sk, _):
            row0 = r0 + sk * SK
            pltpu.sync_copy(dstb_r.at[pl.ds(row0, SK)], didx)

            def group_body(g, _):
                j0 = g * K
                sds = [
                    pltpu.async_copy(ones_v, agg.at[didx.at[j0 + r]],
                                     sem, add=True)
                    for r in range(K)
                ]
                for sd in sds:
                    sd.wait()
                return 0

            return lax.fori_loop(0, NG, group_body, 0)

        lax.fori_loop(0, NSK, super_body, 0)
        plsc.subcore_barrier()
        pltpu.sync_copy(agg.at[pl.ds(st, STRIPE)],
                        out_r.at[cid, pl.ds(st, STRIPE)])

    k = pl.kernel(
        body,
        out_type=jax.ShapeDtypeStruct((NSC, NP, 8), jnp.float32),
        mesh=_sc_mesh(),
        scratch_types=[
            pltpu.VMEM((SK, B), jnp.int32),
            pltpu.VMEM((B, 8), jnp.float32),
            pltpu.VMEM_SHARED((NP, 8), jnp.float32),
            pltpu.SemaphoreType.DMA,
        ],
        compiler_params=pltpu.CompilerParams(use_tc_tiling_on_sc=False),
    )
    return k(dstb, ones, zeros)


def _dinv_of(deg_ref):
    return lax.rsqrt(deg_ref[0, :, 0:1] + deg_ref[1, :, 0:1] + 1.0)


def _mm_body(nc, fc, x_ref, w_ref, deg_ref, *out_refs):
    dinv = _dinv_of(deg_ref)
    h = jnp.dot(x_ref[...], w_ref[...], preferred_element_type=jnp.float32,
                precision=lax.Precision.HIGHEST) * dinv
    for c in range(nc):
        out_refs[c][...] = h[:, c * fc:(c + 1) * fc]


@functools.partial(jax.jit, static_argnums=(0, 1))
def _tc_matmul(nc, fc, x, w, deg):
    fin = x.shape[1]
    f = w.shape[1]
    return pl.pallas_call(
        functools.partial(_mm_body, nc, fc),
        grid=(_NGRID,),
        in_specs=[
            pl.BlockSpec((_NB, fin), lambda i: (i, 0)),
            pl.BlockSpec((fin, f), lambda i: (0, 0)),
            pl.BlockSpec((NSC, _NB, 8), lambda i: (0, i, 0)),
        ],
        out_specs=[pl.BlockSpec((_NB, fc), lambda i: (i, 0))] * nc,
        out_shape=[jax.ShapeDtypeStruct((N, fc), jnp.float32)] * nc,
    )(x, w, deg)


def _combine_body_multi(nc, agg_ref, hp_ref, deg_ref, b_ref, out_ref):
    dinv = _dinv_of(deg_ref)
    parts = [
        dinv * (agg_ref[c, 0] + agg_ref[c, 1] + hp_ref[c])
        for c in range(nc)
    ]
    t = (parts[0] if nc == 1 else jnp.concatenate(parts, axis=1)) + b_ref[...]
    out_ref[...] = jnp.maximum(t, 0.0)


@functools.partial(jax.jit, static_argnums=(0, 1))
def _tc_combine2(nc, fc, agg, hpc, deg, b):
    f = nc * fc
    return pl.pallas_call(
        functools.partial(_combine_body_multi, nc),
        grid=(_NGRID,),
        in_specs=[
            pl.BlockSpec((nc, NSC, _NB, fc), lambda i: (0, 0, i, 0)),
            pl.BlockSpec((nc, _NB, fc), lambda i: (0, i, 0)),
            pl.BlockSpec((NSC, _NB, 8), lambda i: (0, i, 0)),
            pl.BlockSpec((1, f), lambda i: (0, 0)),
        ],
        out_specs=pl.BlockSpec((_NB, f), lambda i: (i, 0)),
        out_shape=jax.ShapeDtypeStruct((N, f), jnp.float32),
    )(agg, hpc, deg, b.reshape(1, f))


def _final_body(agg_ref, hp_ref, deg_ref, b_ref, out_ref):
    dinv = _dinv_of(deg_ref)
    t = dinv * (agg_ref[0, 0] + agg_ref[0, 1] + hp_ref[...]) + b_ref[...]
    t = jnp.maximum(t, 0.0)
    s = jnp.sum(t, axis=1, keepdims=True)
    out_ref[...] = jax.nn.sigmoid(s)


@jax.jit
def _tc_final(agg, hp, deg, b):
    return pl.pallas_call(
        _final_body,
        grid=(_NGRID,),
        in_specs=[
            pl.BlockSpec((1, NSC, _NB, 8), lambda i: (0, 0, i, 0)),
            pl.BlockSpec((_NB, 8), lambda i: (i, 0)),
            pl.BlockSpec((NSC, _NB, 8), lambda i: (0, i, 0)),
            pl.BlockSpec((1, 8), lambda i: (0, 0)),
        ],
        out_specs=pl.BlockSpec((_NB, 1), lambda i: (i, 0)),
        out_shape=jax.ShapeDtypeStruct((N, 1), jnp.float32),
    )(agg, hp, deg, b.reshape(1, 8))


def kernel(value, edge_index, W1, b1, W2, b2, W3, b3, W4, b4, W5, b5):
    src = edge_index[0]
    dst = edge_index[1]
    npad = EP - E
    srcb = jnp.concatenate([src, jnp.zeros((npad,), jnp.int32)]).reshape(RT, B)
    dstb = jnp.concatenate([dst, jnp.full((npad,), N, jnp.int32)]).reshape(RT, B)

    ones8 = jnp.ones((B, 8), jnp.float32)
    zeros8 = jnp.zeros((NP, 8), jnp.float32)
    deg = _sc_degree(dstb, ones8, zeros8)

    W5p = jnp.concatenate([W5, jnp.zeros((W5.shape[0], 4), jnp.float32)], 1)
    b5p = jnp.concatenate([b5, jnp.zeros((4,), jnp.float32)])
    ws = [W1, W2, W3, W4, W5p]
    bs = [b1, b2, b3, b4, b5p]
    x = value
    for l in range(5):
        f = _DIMS[l + 1] if l < 4 else 8
        nc, fc = _chunking(f)
        hps = _tc_matmul(nc, fc, x, ws[l], deg)
        zeros = jnp.zeros((NP, fc), jnp.float32)
        agg = _sc_scatter(nc, fc, hps, srcb, dstb, zeros)
        if l < 4:
            hpc = jnp.stack(hps, axis=0)
            x = _tc_combine2(nc, fc, agg, hpc, deg, bs[l])
        else:
            out = _tc_final(agg, hps[0], deg, bs[l])
    return out.reshape(N)


# speedup vs baseline: 25.9064x; 1.1532x over previous
import functools

import jax
import jax.numpy as jnp
from jax import lax
from jax.experimental import pallas as pl
from jax.experimental.pallas import tpu as pltpu
from jax.experimental.pallas import tpu_sc as plsc

N = 100000
E = 3200000
B = 128
NSC = 2
NSUB = 16
NW = NSC * NSUB
RT = 25088
RPW = RT // NW
EP = RT * B
NP = 100096
STRIPE = NP // NSUB
SK = 56
NSK = RPW // SK
K = 8
NG = SK // K
DSK = 112
DNSK = RPW // DSK
DK = 16
DNG = DSK // DK

_DIMS = [21, 8, 16, 32, 64, 8]
_NB = 2000
_NGRID = N // _NB


def _chunking(f):
    fc = f if f < 16 else 16
    return f // fc, fc


def _sc_mesh():
    return plsc.VectorSubcoreMesh(core_axis_name="c", subcore_axis_name="s")


@functools.partial(jax.jit, static_argnums=(0, 1))
def _sc_scatter(nc, fc, hps, srcb, dstb, zeros):
    def body(*refs):
        hp_refs = refs[:nc]
        srcb_r, dstb_r, zeros_r, out_r = refs[nc:nc + 4]
        sidx, didx, vals, agg, gsem, ssem = refs[nc + 4:]
        cid = lax.axis_index("c")
        sid = lax.axis_index("s")
        wid = cid * NSUB + sid
        r0 = wid * RPW
        st = sid * STRIPE
        for c in range(nc):
            pltpu.sync_copy(zeros_r.at[pl.ds(st, STRIPE)],
                            agg.at[pl.ds(st, STRIPE)])
            plsc.subcore_barrier()

            def super_body(sk, _, c=c):
                row0 = r0 + sk * SK
                pltpu.sync_copy(srcb_r.at[pl.ds(row0, SK)], sidx)
                pltpu.sync_copy(dstb_r.at[pl.ds(row0, SK)], didx)

                def group_body(g, _):
                    j0 = g * K
                    gds = [
                        pltpu.async_copy(hp_refs[c].at[sidx.at[j0 + r]],
                                         vals.at[r], gsem)
                        for r in range(K)
                    ]
                    for gd in gds:
                        gd.wait()
                    sds = [
                        pltpu.async_copy(vals.at[r], agg.at[didx.at[j0 + r]],
                                         ssem, add=True)
                        for r in range(K)
                    ]
                    for sd in sds:
                        sd.wait()
                    return 0

                return lax.fori_loop(0, NG, group_body, 0)

            lax.fori_loop(0, NSK, super_body, 0)
            plsc.subcore_barrier()
            pltpu.sync_copy(agg.at[pl.ds(st, STRIPE)],
                            out_r.at[c, cid, pl.ds(st, STRIPE)])
            plsc.subcore_barrier()

    k = pl.kernel(
        body,
        out_type=jax.ShapeDtypeStruct((nc, NSC, NP, fc), jnp.float32),
        mesh=_sc_mesh(),
        scratch_types=[
            pltpu.VMEM((SK, B), jnp.int32),
            pltpu.VMEM((SK, B), jnp.int32),
            pltpu.VMEM((K, B, fc), jnp.float32),
            pltpu.VMEM_SHARED((NP, fc), jnp.float32),
            pltpu.SemaphoreType.DMA,
            pltpu.SemaphoreType.DMA,
        ],
        compiler_params=pltpu.CompilerParams(use_tc_tiling_on_sc=False),
    )
    return k(*hps, srcb, dstb, zeros)


@jax.jit
def _sc_degree(dstb, ones, zeros):
    def body(dstb_r, ones_r, zeros_r, out_r, didx, ones_v, agg, sem):
        cid = lax.axis_index("c")
        sid = lax.axis_index("s")
        wid = cid * NSUB + sid
        r0 = wid * RPW
        st = sid * STRIPE
        pltpu.sync_copy(ones_r, ones_v)
        pltpu.sync_copy(zeros_r.at[pl.ds(st, STRIPE)],
                        agg.at[pl.ds(st, STRIPE)])
        plsc.subcore_barrier()

        def super_body(sk, _):
            row0 = r0 + sk * DSK
            pltpu.sync_copy(dstb_r.at[pl.ds(row0, DSK)], didx)

            def group_body(g, _):
                j0 = g * DK
                sds = [
                    pltpu.async_copy(ones_v, agg.at[didx.at[j0 + r]],
                                     sem, add=True)
                    for r in range(DK)
                ]
                for sd in sds:
                    sd.wait()
                return 0

            return lax.fori_loop(0, DNG, group_body, 0)

        lax.fori_loop(0, DNSK, super_body, 0)
        plsc.subcore_barrier()
        pltpu.sync_copy(agg.at[pl.ds(st, STRIPE)],
                        out_r.at[cid, pl.ds(st, STRIPE)])

    k = pl.kernel(
        body,
        out_type=jax.ShapeDtypeStruct((NSC, NP, 8), jnp.float32),
        mesh=_sc_mesh(),
        scratch_types=[
            pltpu.VMEM((DSK, B), jnp.int32),
            pltpu.VMEM((B, 8), jnp.float32),
            pltpu.VMEM_SHARED((NP, 8), jnp.float32),
            pltpu.SemaphoreType.DMA,
        ],
        compiler_params=pltpu.CompilerParams(use_tc_tiling_on_sc=False),
    )
    return k(dstb, ones, zeros)


def _prep_body(deg_ref, out_ref):
    out_ref[...] = lax.rsqrt(deg_ref[0, :, 0:1] + deg_ref[1, :, 0:1] + 1.0)


@jax.jit
def _tc_prep(deg):
    return pl.pallas_call(
        _prep_body,
        grid=(_NGRID,),
        in_specs=[pl.BlockSpec((NSC, _NB, 8), lambda i: (0, i, 0))],
        out_specs=pl.BlockSpec((_NB, 1), lambda i: (i, 0)),
        out_shape=jax.ShapeDtypeStruct((N, 1), jnp.float32),
    )(deg)


def _small_matmul(x, w):
    return jnp.dot(x, w, preferred_element_type=jnp.float32,
                   precision=lax.Precision.HIGHEST)


def _mm_body(nc, fc, x_ref, w_ref, dinv_ref, *out_refs):
    h = _small_matmul(x_ref[...], w_ref[...]) * dinv_ref[...]
    for c in range(nc):
        out_refs[c][...] = h[:, c * fc:(c + 1) * fc]


@functools.partial(jax.jit, static_argnums=(0, 1))
def _tc_matmul(nc, fc, x, w, dinv):
    fin = x.shape[1]
    f = w.shape[1]
    return pl.pallas_call(
        functools.partial(_mm_body, nc, fc),
        grid=(_NGRID,),
        in_specs=[
            pl.BlockSpec((_NB, fin), lambda i: (i, 0)),
            pl.BlockSpec((fin, f), lambda i: (0, 0)),
            pl.BlockSpec((_NB, 1), lambda i: (i, 0)),
        ],
        out_specs=[pl.BlockSpec((_NB, fc), lambda i: (i, 0))] * nc,
        out_shape=[jax.ShapeDtypeStruct((N, fc), jnp.float32)] * nc,
    )(x, w, dinv)


def _fuse_body(ncl, fcl, ncn, fcn, agg_ref, *refs):
    hp_refs = refs[:ncl]
    dinv_ref, b_ref, w_ref = refs[ncl:ncl + 3]
    out_refs = refs[ncl + 3:]
    dinv = dinv_ref[...]
    b = b_ref[...]
    parts = [
        dinv * (agg_ref[c, 0] + agg_ref[c, 1] + hp_refs[c][...])
        + b[:, c * fcl:(c + 1) * fcl]
        for c in range(ncl)
    ]
    x = parts[0] if ncl == 1 else jnp.concatenate(parts, axis=1)
    x = jnp.maximum(x, 0.0)
    h = _small_matmul(x, w_ref[...]) * dinv
    for c in range(ncn):
        out_refs[c][...] = h[:, c * fcn:(c + 1) * fcn]


@functools.partial(jax.jit, static_argnums=(0, 1, 2, 3))
def _tc_fuse(ncl, fcl, ncn, fcn, agg, hps, dinv, b, w):
    fl = ncl * fcl
    fn = w.shape[1]
    return pl.pallas_call(
        functools.partial(_fuse_body, ncl, fcl, ncn, fcn),
        grid=(_NGRID,),
        in_specs=[
            pl.BlockSpec((ncl, NSC, _NB, fcl), lambda i: (0, 0, i, 0)),
        ] + [
            pl.BlockSpec((_NB, fcl), lambda i: (i, 0)),
        ] * ncl + [
            pl.BlockSpec((_NB, 1), lambda i: (i, 0)),
            pl.BlockSpec((1, fl), lambda i: (0, 0)),
            pl.BlockSpec((fl, fn), lambda i: (0, 0)),
        ],
        out_specs=[pl.BlockSpec((_NB, fcn), lambda i: (i, 0))] * ncn,
        out_shape=[jax.ShapeDtypeStruct((N, fcn), jnp.float32)] * ncn,
    )(agg, *hps, dinv, b.reshape(1, fl), w)


def _final_body(agg_ref, hp_ref, dinv_ref, b_ref, out_ref):
    dinv = dinv_ref[...]
    t = dinv * (agg_ref[0, 0] + agg_ref[0, 1] + hp_ref[...]) + b_ref[...]
    t = jnp.maximum(t, 0.0)
    s = jnp.sum(t, axis=1, keepdims=True)
    out_ref[...] = jax.nn.sigmoid(s)


@jax.jit
def _tc_final(agg, hp, dinv, b):
    return pl.pallas_call(
        _final_body,
        grid=(_NGRID,),
        in_specs=[
            pl.BlockSpec((1, NSC, _NB, 8), lambda i: (0, 0, i, 0)),
            pl.BlockSpec((_NB, 8), lambda i: (i, 0)),
            pl.BlockSpec((_NB, 1), lambda i: (i, 0)),
            pl.BlockSpec((1, 8), lambda i: (0, 0)),
        ],
        out_specs=pl.BlockSpec((_NB, 1), lambda i: (i, 0)),
        out_shape=jax.ShapeDtypeStruct((N, 1), jnp.float32),
    )(agg, hp, dinv, b.reshape(1, 8))


def kernel(value, edge_index, W1, b1, W2, b2, W3, b3, W4, b4, W5, b5):
    src = edge_index[0]
    dst = edge_index[1]
    npad = EP - E
    srcb = jnp.concatenate([src, jnp.zeros((npad,), jnp.int32)]).reshape(RT, B)
    dstb = jnp.concatenate([dst, jnp.full((npad,), N, jnp.int32)]).reshape(RT, B)

    ones8 = jnp.ones((B, 8), jnp.float32)
    zeros8 = jnp.zeros((NP, 8), jnp.float32)
    deg = _sc_degree(dstb, ones8, zeros8)
    dinv = _tc_prep(deg)

    W5p = jnp.concatenate([W5, jnp.zeros((W5.shape[0], 4), jnp.float32)], 1)
    b5p = jnp.concatenate([b5, jnp.zeros((4,), jnp.float32)])
    ws = [W1, W2, W3, W4, W5p]
    bs = [b1, b2, b3, b4, b5p]

    nc, fc = _chunking(_DIMS[1])
    hps = _tc_matmul(nc, fc, value, ws[0], dinv)
    for l in range(5):
        zeros = jnp.zeros((NP, fc), jnp.float32)
        agg = _sc_scatter(nc, fc, hps, srcb, dstb, zeros)
        if l < 4:
            ncn, fcn = _chunking(_DIMS[l + 2])
            hps = _tc_fuse(nc, fc, ncn, fcn, agg, hps, dinv, bs[l], ws[l + 1])
            nc, fc = ncn, fcn
        else:
            out = _tc_final(agg, hps[0], dinv, bs[l])
    return out.reshape(N)


# speedup vs baseline: 28.6749x; 1.1069x over previous
import functools

import jax
import jax.numpy as jnp
from jax import lax
from jax.experimental import pallas as pl
from jax.experimental.pallas import tpu as pltpu
from jax.experimental.pallas import tpu_sc as plsc

N = 100000
E = 3200000
B = 128
NSC = 2
NSUB = 16
NW = NSC * NSUB
RT = 25088
RPW = RT // NW
EP = RT * B
NP = 100096
STRIPE = NP // NSUB
SK = 56
NSK = RPW // SK
K4 = 4
NG = SK // K4
NPAIR = NG // 2
DSK = 112
DNSK = RPW // DSK
DK = 16
DNG = DSK // DK

_DIMS = [21, 8, 16, 32, 64, 8]
_NB = 2000
_NGRID = N // _NB


def _chunking(f):
    fc = f if f < 16 else 16
    return f // fc, fc


def _sc_mesh():
    return plsc.VectorSubcoreMesh(core_axis_name="c", subcore_axis_name="s")


@functools.partial(jax.jit, static_argnums=(0, 1))
def _sc_scatter(nc, fc, hps, srcb, dstb, zeros):
    def body(*refs):
        hp_refs = refs[:nc]
        srcb_r, dstb_r, zeros_r, out_r = refs[nc:nc + 4]
        sidx, didx, vals, agg, gsem, ssem = refs[nc + 4:]
        cid = lax.axis_index("c")
        sid = lax.axis_index("s")
        wid = cid * NSUB + sid
        r0 = wid * RPW
        st = sid * STRIPE
        for c in range(nc):
            pltpu.sync_copy(zeros_r.at[pl.ds(st, STRIPE)],
                            agg.at[pl.ds(st, STRIPE)])
            plsc.subcore_barrier()

            def super_body(sk, _, c=c):
                row0 = r0 + sk * SK
                pltpu.sync_copy(srcb_r.at[pl.ds(row0, SK)], sidx)
                pltpu.sync_copy(dstb_r.at[pl.ds(row0, SK)], didx)

                def fire_gathers(s, j0):
                    for r in range(K4):
                        pltpu.async_copy(hp_refs[c].at[sidx.at[j0 + r]],
                                         vals.at[s, r], gsem)

                def drain_gathers(s):
                    for r in range(K4):
                        pltpu.make_async_copy(hp_refs[c].at[sidx.at[0]],
                                              vals.at[s, r], gsem).wait()

                def do_scatters(s, j0):
                    sds = [
                        pltpu.async_copy(vals.at[s, r],
                                         agg.at[didx.at[j0 + r]],
                                         ssem, add=True)
                        for r in range(K4)
                    ]
                    for sd in sds:
                        sd.wait()

                fire_gathers(0, 0)

                def pair_body(p, _):
                    j0 = 2 * p * K4
                    fire_gathers(1, j0 + K4)
                    drain_gathers(0)
                    do_scatters(0, j0)
                    fire_gathers(0, j0 + 2 * K4)
                    drain_gathers(1)
                    do_scatters(1, j0 + K4)
                    return 0

                lax.fori_loop(0, NPAIR - 1, pair_body, 0)
                j0 = (NG - 2) * K4
                fire_gathers(1, j0 + K4)
                drain_gathers(0)
                do_scatters(0, j0)
                drain_gathers(1)
                do_scatters(1, j0 + K4)
                return 0

            lax.fori_loop(0, NSK, super_body, 0)
            plsc.subcore_barrier()
            pltpu.sync_copy(agg.at[pl.ds(st, STRIPE)],
                            out_r.at[c, cid, pl.ds(st, STRIPE)])
            plsc.subcore_barrier()

    k = pl.kernel(
        body,
        out_type=jax.ShapeDtypeStruct((nc, NSC, NP, fc), jnp.float32),
        mesh=_sc_mesh(),
        scratch_types=[
            pltpu.VMEM((SK, B), jnp.int32),
            pltpu.VMEM((SK, B), jnp.int32),
            pltpu.VMEM((2, K4, B, fc), jnp.float32),
            pltpu.VMEM_SHARED((NP, fc), jnp.float32),
            pltpu.SemaphoreType.DMA,
            pltpu.SemaphoreType.DMA,
        ],
        compiler_params=pltpu.CompilerParams(use_tc_tiling_on_sc=False),
    )
    return k(*hps, srcb, dstb, zeros)


@jax.jit
def _sc_degree(dstb, ones, zeros):
    def body(dstb_r, ones_r, zeros_r, out_r, didx, ones_v, agg, sem):
        cid = lax.axis_index("c")
        sid = lax.axis_index("s")
        wid = cid * NSUB + sid
        r0 = wid * RPW
        st = sid * STRIPE
        pltpu.sync_copy(ones_r, ones_v)
        pltpu.sync_copy(zeros_r.at[pl.ds(st, STRIPE)],
                        agg.at[pl.ds(st, STRIPE)])
        plsc.subcore_barrier()

        def super_body(sk, _):
            row0 = r0 + sk * DSK
            pltpu.sync_copy(dstb_r.at[pl.ds(row0, DSK)], didx)

            def group_body(g, _):
                j0 = g * DK
                sds = [
                    pltpu.async_copy(ones_v, agg.at[didx.at[j0 + r]],
                                     sem, add=True)
                    for r in range(DK)
                ]
                for sd in sds:
                    sd.wait()
                return 0

            return lax.fori_loop(0, DNG, group_body, 0)

        lax.fori_loop(0, DNSK, super_body, 0)
        plsc.subcore_barrier()
        pltpu.sync_copy(agg.at[pl.ds(st, STRIPE)],
                        out_r.at[cid, pl.ds(st, STRIPE)])

    k = pl.kernel(
        body,
        out_type=jax.ShapeDtypeStruct((NSC, NP, 8), jnp.float32),
        mesh=_sc_mesh(),
        scratch_types=[
            pltpu.VMEM((DSK, B), jnp.int32),
            pltpu.VMEM((B, 8), jnp.float32),
            pltpu.VMEM_SHARED((NP, 8), jnp.float32),
            pltpu.SemaphoreType.DMA,
        ],
        compiler_params=pltpu.CompilerParams(use_tc_tiling_on_sc=False),
    )
    return k(dstb, ones, zeros)


def _prep_body(deg_ref, out_ref):
    out_ref[...] = lax.rsqrt(deg_ref[0, :, 0:1] + deg_ref[1, :, 0:1] + 1.0)


@jax.jit
def _tc_prep(deg):
    return pl.pallas_call(
        _prep_body,
        grid=(_NGRID,),
        in_specs=[pl.BlockSpec((NSC, _NB, 8), lambda i: (0, i, 0))],
        out_specs=pl.BlockSpec((_NB, 1), lambda i: (i, 0)),
        out_shape=jax.ShapeDtypeStruct((N, 1), jnp.float32),
    )(deg)


def _small_matmul(x, w):
    return jnp.dot(x, w, preferred_element_type=jnp.float32,
                   precision=lax.Precision.HIGHEST)


def _mm_body(nc, fc, x_ref, w_ref, dinv_ref, *out_refs):
    h = _small_matmul(x_ref[...], w_ref[...]) * dinv_ref[...]
    for c in range(nc):
        out_refs[c][...] = h[:, c * fc:(c + 1) * fc]


@functools.partial(jax.jit, static_argnums=(0, 1))
def _tc_matmul(nc, fc, x, w, dinv):
    fin = x.shape[1]
    f = w.shape[1]
    return pl.pallas_call(
        functools.partial(_mm_body, nc, fc),
        grid=(_NGRID,),
        in_specs=[
            pl.BlockSpec((_NB, fin), lambda i: (i, 0)),
            pl.BlockSpec((fin, f), lambda i: (0, 0)),
            pl.BlockSpec((_NB, 1), lambda i: (i, 0)),
        ],
        out_specs=[pl.BlockSpec((_NB, fc), lambda i: (i, 0))] * nc,
        out_shape=[jax.ShapeDtypeStruct((N, fc), jnp.float32)] * nc,
    )(x, w, dinv)


def _fuse_body(ncl, fcl, ncn, fcn, agg_ref, *refs):
    hp_refs = refs[:ncl]
    dinv_ref, b_ref, w_ref = refs[ncl:ncl + 3]
    out_refs = refs[ncl + 3:]
    dinv = dinv_ref[...]
    b = b_ref[...]
    parts = [
        dinv * (agg_ref[c, 0] + agg_ref[c, 1] + hp_refs[c][...])
        + b[:, c * fcl:(c + 1) * fcl]
        for c in range(ncl)
    ]
    x = parts[0] if ncl == 1 else jnp.concatenate(parts, axis=1)
    x = jnp.maximum(x, 0.0)
    h = _small_matmul(x, w_ref[...]) * dinv
    for c in range(ncn):
        out_refs[c][...] = h[:, c * fcn:(c + 1) * fcn]


@functools.partial(jax.jit, static_argnums=(0, 1, 2, 3))
def _tc_fuse(ncl, fcl, ncn, fcn, agg, hps, dinv, b, w):
    fl = ncl * fcl
    fn = w.shape[1]
    return pl.pallas_call(
        functools.partial(_fuse_body, ncl, fcl, ncn, fcn),
        grid=(_NGRID,),
        in_specs=[
            pl.BlockSpec((ncl, NSC, _NB, fcl), lambda i: (0, 0, i, 0)),
        ] + [
            pl.BlockSpec((_NB, fcl), lambda i: (i, 0)),
        ] * ncl + [
            pl.BlockSpec((_NB, 1), lambda i: (i, 0)),
            pl.BlockSpec((1, fl), lambda i: (0, 0)),
            pl.BlockSpec((fl, fn), lambda i: (0, 0)),
        ],
        out_specs=[pl.BlockSpec((_NB, fcn), lambda i: (i, 0))] * ncn,
        out_shape=[jax.ShapeDtypeStruct((N, fcn), jnp.float32)] * ncn,
    )(agg, *hps, dinv, b.reshape(1, fl), w)


def _final_body(agg_ref, hp_ref, dinv_ref, b_ref, out_ref):
    dinv = dinv_ref[...]
    t = dinv * (agg_ref[0, 0] + agg_ref[0, 1] + hp_ref[...]) + b_ref[...]
    t = jnp.maximum(t, 0.0)
    s = jnp.sum(t, axis=1, keepdims=True)
    out_ref[...] = jax.nn.sigmoid(s)


@jax.jit
def _tc_final(agg, hp, dinv, b):
    return pl.pallas_call(
        _final_body,
        grid=(_NGRID,),
        in_specs=[
            pl.BlockSpec((1, NSC, _NB, 8), lambda i: (0, 0, i, 0)),
            pl.BlockSpec((_NB, 8), lambda i: (i, 0)),
            pl.BlockSpec((_NB, 1), lambda i: (i, 0)),
            pl.BlockSpec((1, 8), lambda i: (0, 0)),
        ],
        out_specs=pl.BlockSpec((_NB, 1), lambda i: (i, 0)),
        out_shape=jax.ShapeDtypeStruct((N, 1), jnp.float32),
    )(agg, hp, dinv, b.reshape(1, 8))


def kernel(value, edge_index, W1, b1, W2, b2, W3, b3, W4, b4, W5, b5):
    src = edge_index[0]
    dst = edge_index[1]
    npad = EP - E
    srcb = jnp.concatenate([src, jnp.zeros((npad,), jnp.int32)]).reshape(RT, B)
    dstb = jnp.concatenate([dst, jnp.full((npad,), N, jnp.int32)]).reshape(RT, B)

    ones8 = jnp.ones((B, 8), jnp.float32)
    zeros8 = jnp.zeros((NP, 8), jnp.float32)
    deg = _sc_degree(dstb, ones8, zeros8)
    dinv = _tc_prep(deg)

    W5p = jnp.concatenate([W5, jnp.zeros((W5.shape[0], 4), jnp.float32)], 1)
    b5p = jnp.concatenate([b5, jnp.zeros((4,), jnp.float32)])
    ws = [W1, W2, W3, W4, W5p]
    bs = [b1, b2, b3, b4, b5p]

    nc, fc = _chunking(_DIMS[1])
    hps = _tc_matmul(nc, fc, value, ws[0], dinv)
    for l in range(5):
        zeros = jnp.zeros((NP, fc), jnp.float32)
        agg = _sc_scatter(nc, fc, hps, srcb, dstb, zeros)
        if l < 4:
            ncn, fcn = _chunking(_DIMS[l + 2])
            hps = _tc_fuse(nc, fc, ncn, fcn, agg, hps, dinv, bs[l], ws[l + 1])
            nc, fc = ncn, fcn
        else:
            out = _tc_final(agg, hps[0], dinv, bs[l])
    return out.reshape(N)


# speedup vs baseline: 32.7000x; 1.1404x over previous
import functools

import jax
import jax.numpy as jnp
from jax import lax
from jax.experimental import pallas as pl
from jax.experimental.pallas import tpu as pltpu
from jax.experimental.pallas import tpu_sc as plsc

N = 100000
E = 3200000
B = 128
NSC = 2
NSUB = 16
NW = NSC * NSUB
RT = 25088
RPW = RT // NW
EP = RT * B
NP = 100096
STRIPE = NP // NSUB
SK = 56
NSK = RPW // SK
K4 = 4
NG = SK // K4
NPAIR = NG // 2
DSK = 112
DNSK = RPW // DSK
DK = 16
DNG = DSK // DK

_DIMS = [21, 8, 16, 32, 64, 8]
_NB = 2000
_NGRID = N // _NB


def _chunking(f):
    fc = f if f < 16 else 16
    return f // fc, fc


def _sc_mesh():
    return plsc.VectorSubcoreMesh(core_axis_name="c", subcore_axis_name="s")


@functools.partial(jax.jit, static_argnums=(0, 1))
def _sc_scatter(nc, fc, hps, srcb, dstb, zeros):
    def body(*refs):
        hp_refs = refs[:nc]
        srcb_r, dstb_r, zeros_r, out_r = refs[nc:nc + 4]
        sidx, didx, vals, agg, gsem, ssem = refs[nc + 4:]
        cid = lax.axis_index("c")
        sid = lax.axis_index("s")
        wid = cid * NSUB + sid
        r0 = wid * RPW
        st = sid * STRIPE
        for c in range(nc):
            pltpu.sync_copy(zeros_r.at[pl.ds(st, STRIPE)],
                            agg.at[pl.ds(st, STRIPE)])
            plsc.subcore_barrier()

            def super_body(sk, _, c=c):
                row0 = r0 + sk * SK
                pltpu.sync_copy(srcb_r.at[pl.ds(row0, SK)], sidx)
                pltpu.sync_copy(dstb_r.at[pl.ds(row0, SK)], didx)

                def fire_gathers(s, j0):
                    for r in range(K4):
                        pltpu.async_copy(hp_refs[c].at[sidx.at[j0 + r]],
                                         vals.at[s, r], gsem)

                def drain_gathers(s):
                    for r in range(K4):
                        pltpu.make_async_copy(hp_refs[c].at[sidx.at[0]],
                                              vals.at[s, r], gsem).wait()

                def do_scatters(s, j0):
                    sds = [
                        pltpu.async_copy(vals.at[s, r],
                                         agg.at[didx.at[j0 + r]],
                                         ssem, add=True)
                        for r in range(K4)
                    ]
                    for sd in sds:
                        sd.wait()

                fire_gathers(0, 0)

                def pair_body(p, _):
                    j0 = 2 * p * K4
                    fire_gathers(1, j0 + K4)
                    drain_gathers(0)
                    do_scatters(0, j0)
                    fire_gathers(0, j0 + 2 * K4)
                    drain_gathers(1)
                    do_scatters(1, j0 + K4)
                    return 0

                lax.fori_loop(0, NPAIR - 1, pair_body, 0)
                j0 = (NG - 2) * K4
                fire_gathers(1, j0 + K4)
                drain_gathers(0)
                do_scatters(0, j0)
                drain_gathers(1)
                do_scatters(1, j0 + K4)
                return 0

            lax.fori_loop(0, NSK, super_body, 0)
            plsc.subcore_barrier()
            pltpu.sync_copy(agg.at[pl.ds(st, STRIPE)],
                            out_r.at[pl.ds(st, STRIPE),
                                     pl.ds(cid * 64 + c * fc, fc)])
            plsc.subcore_barrier()

    k = pl.kernel(
        body,
        out_type=jax.ShapeDtypeStruct((NP, 128), jnp.float32),
        mesh=_sc_mesh(),
        scratch_types=[
            pltpu.VMEM((SK, B), jnp.int32),
            pltpu.VMEM((SK, B), jnp.int32),
            pltpu.VMEM((2, K4, B, fc), jnp.float32),
            pltpu.VMEM_SHARED((NP, fc), jnp.float32),
            pltpu.SemaphoreType.DMA,
            pltpu.SemaphoreType.DMA,
        ],
        compiler_params=pltpu.CompilerParams(use_tc_tiling_on_sc=False),
    )
    return k(*hps, srcb, dstb, zeros)


@jax.jit
def _sc_degree(dstb, ones, zeros):
    def body(dstb_r, ones_r, zeros_r, out_r, didx, ones_v, agg, sem):
        cid = lax.axis_index("c")
        sid = lax.axis_index("s")
        wid = cid * NSUB + sid
        r0 = wid * RPW
        st = sid * STRIPE
        pltpu.sync_copy(ones_r, ones_v)
        pltpu.sync_copy(zeros_r.at[pl.ds(st, STRIPE)],
                        agg.at[pl.ds(st, STRIPE)])
        plsc.subcore_barrier()

        def super_body(sk, _):
            row0 = r0 + sk * DSK
            pltpu.sync_copy(dstb_r.at[pl.ds(row0, DSK)], didx)

            def group_body(g, _):
                j0 = g * DK
                sds = [
                    pltpu.async_copy(ones_v, agg.at[didx.at[j0 + r]],
                                     sem, add=True)
                    for r in range(DK)
                ]
                for sd in sds:
                    sd.wait()
                return 0

            return lax.fori_loop(0, DNG, group_body, 0)

        lax.fori_loop(0, DNSK, super_body, 0)
        plsc.subcore_barrier()
        pltpu.sync_copy(agg.at[pl.ds(st, STRIPE)],
                        out_r.at[cid, pl.ds(st, STRIPE)])

    k = pl.kernel(
        body,
        out_type=jax.ShapeDtypeStruct((NSC, NP, 8), jnp.float32),
        mesh=_sc_mesh(),
        scratch_types=[
            pltpu.VMEM((DSK, B), jnp.int32),
            pltpu.VMEM((B, 8), jnp.float32),
            pltpu.VMEM_SHARED((NP, 8), jnp.float32),
            pltpu.SemaphoreType.DMA,
        ],
        compiler_params=pltpu.CompilerParams(use_tc_tiling_on_sc=False),
    )
    return k(dstb, ones, zeros)


def _prep_body(deg_ref, out_ref):
    out_ref[...] = lax.rsqrt(deg_ref[0, :, 0:1] + deg_ref[1, :, 0:1] + 1.0)


@jax.jit
def _tc_prep(deg):
    return pl.pallas_call(
        _prep_body,
        grid=(_NGRID,),
        in_specs=[pl.BlockSpec((NSC, _NB, 8), lambda i: (0, i, 0))],
        out_specs=pl.BlockSpec((_NB, 1), lambda i: (i, 0)),
        out_shape=jax.ShapeDtypeStruct((N, 1), jnp.float32),
    )(deg)


def _small_matmul(x, w):
    return jnp.dot(x, w, preferred_element_type=jnp.float32,
                   precision=lax.Precision.HIGHEST)


def _mm_body(nc, fc, x_ref, w_ref, dinv_ref, *out_refs):
    h = _small_matmul(x_ref[...], w_ref[...]) * dinv_ref[...]
    for c in range(nc):
        out_refs[c][...] = h[:, c * fc:(c + 1) * fc]


@functools.partial(jax.jit, static_argnums=(0, 1))
def _tc_matmul(nc, fc, x, w, dinv):
    fin = x.shape[1]
    f = w.shape[1]
    return pl.pallas_call(
        functools.partial(_mm_body, nc, fc),
        grid=(_NGRID,),
        in_specs=[
            pl.BlockSpec((_NB, fin), lambda i: (i, 0)),
            pl.BlockSpec((fin, f), lambda i: (0, 0)),
            pl.BlockSpec((_NB, 1), lambda i: (i, 0)),
        ],
        out_specs=[pl.BlockSpec((_NB, fc), lambda i: (i, 0))] * nc,
        out_shape=[jax.ShapeDtypeStruct((N, fc), jnp.float32)] * nc,
    )(x, w, dinv)


def _fuse_body(ncl, fcl, ncn, fcn, agg_ref, *refs):
    hp_refs = refs[:ncl]
    dinv_ref, b_ref, w_ref = refs[ncl:ncl + 3]
    out_refs = refs[ncl + 3:]
    dinv = dinv_ref[...]
    b = b_ref[...]
    a = agg_ref[...]
    parts = [
        dinv * (a[:, c * fcl:(c + 1) * fcl]
                + a[:, 64 + c * fcl:64 + (c + 1) * fcl]
                + hp_refs[c][...])
        + b[:, c * fcl:(c + 1) * fcl]
        for c in range(ncl)
    ]
    x = parts[0] if ncl == 1 else jnp.concatenate(parts, axis=1)
    x = jnp.maximum(x, 0.0)
    h = _small_matmul(x, w_ref[...]) * dinv
    for c in range(ncn):
        out_refs[c][...] = h[:, c * fcn:(c + 1) * fcn]


@functools.partial(jax.jit, static_argnums=(0, 1, 2, 3))
def _tc_fuse(ncl, fcl, ncn, fcn, agg, hps, dinv, b, w):
    fl = ncl * fcl
    fn = w.shape[1]
    return pl.pallas_call(
        functools.partial(_fuse_body, ncl, fcl, ncn, fcn),
        grid=(_NGRID,),
        in_specs=[
            pl.BlockSpec((_NB, 128), lambda i: (i, 0)),
        ] + [
            pl.BlockSpec((_NB, fcl), lambda i: (i, 0)),
        ] * ncl + [
            pl.BlockSpec((_NB, 1), lambda i: (i, 0)),
            pl.BlockSpec((1, fl), lambda i: (0, 0)),
            pl.BlockSpec((fl, fn), lambda i: (0, 0)),
        ],
        out_specs=[pl.BlockSpec((_NB, fcn), lambda i: (i, 0))] * ncn,
        out_shape=[jax.ShapeDtypeStruct((N, fcn), jnp.float32)] * ncn,
    )(agg, *hps, dinv, b.reshape(1, fl), w)


def _final_body(agg_ref, hp_ref, dinv_ref, b_ref, out_ref):
    dinv = dinv_ref[...]
    a = agg_ref[...]
    t = dinv * (a[:, 0:8] + a[:, 64:72] + hp_ref[...]) + b_ref[...]
    t = jnp.maximum(t, 0.0)
    s = jnp.sum(t, axis=1, keepdims=True)
    out_ref[...] = jax.nn.sigmoid(s)


@jax.jit
def _tc_final(agg, hp, dinv, b):
    return pl.pallas_call(
        _final_body,
        grid=(_NGRID,),
        in_specs=[
            pl.BlockSpec((_NB, 128), lambda i: (i, 0)),
            pl.BlockSpec((_NB, 8), lambda i: (i, 0)),
            pl.BlockSpec((_NB, 1), lambda i: (i, 0)),
            pl.BlockSpec((1, 8), lambda i: (0, 0)),
        ],
        out_specs=pl.BlockSpec((_NB, 1), lambda i: (i, 0)),
        out_shape=jax.ShapeDtypeStruct((N, 1), jnp.float32),
    )(agg, hp, dinv, b.reshape(1, 8))


def kernel(value, edge_index, W1, b1, W2, b2, W3, b3, W4, b4, W5, b5):
    src = edge_index[0]
    dst = edge_index[1]
    npad = EP - E
    srcb = jnp.concatenate([src, jnp.zeros((npad,), jnp.int32)]).reshape(RT, B)
    dstb = jnp.concatenate([dst, jnp.full((npad,), N, jnp.int32)]).reshape(RT, B)

    ones8 = jnp.ones((B, 8), jnp.float32)
    zeros8 = jnp.zeros((NP, 8), jnp.float32)
    deg = _sc_degree(dstb, ones8, zeros8)
    dinv = _tc_prep(deg)

    W5p = jnp.concatenate([W5, jnp.zeros((W5.shape[0], 4), jnp.float32)], 1)
    b5p = jnp.concatenate([b5, jnp.zeros((4,), jnp.float32)])
    ws = [W1, W2, W3, W4, W5p]
    bs = [b1, b2, b3, b4, b5p]

    nc, fc = _chunking(_DIMS[1])
    hps = _tc_matmul(nc, fc, value, ws[0], dinv)
    for l in range(5):
        zeros = jnp.zeros((NP, fc), jnp.float32)
        agg = _sc_scatter(nc, fc, hps, srcb, dstb, zeros)
        if l < 4:
            ncn, fcn = _chunking(_DIMS[l + 2])
            hps = _tc_fuse(nc, fc, ncn, fcn, agg, hps, dinv, bs[l], ws[l + 1])
            nc, fc = ncn, fcn
        else:
            out = _tc_final(agg, hps[0], dinv, bs[l])
    return out.reshape(N)


# speedup vs baseline: 33.0847x; 1.0118x over previous
import functools

import jax
import jax.numpy as jnp
from jax import lax
from jax.experimental import pallas as pl
from jax.experimental.pallas import tpu as pltpu
from jax.experimental.pallas import tpu_sc as plsc

N = 100000
E = 3200000
B = 128
NSC = 2
NSUB = 16
NW = NSC * NSUB
RT = 25088
RPW = RT // NW
EP = RT * B
NP = 100096
STRIPE = NP // NSUB
SK = 56
NSK = RPW // SK
K4 = 4
NG = SK // K4
NPAIR = NG // 2
DSK = 112
DNSK = RPW // DSK
DK = 16
DNG = DSK // DK

_DIMS = [21, 8, 16, 32, 64, 8]
_NB = 5000
_NGRID = N // _NB


def _chunking(f):
    fc = f if f < 16 else 16
    return f // fc, fc


def _sc_mesh():
    return plsc.VectorSubcoreMesh(core_axis_name="c", subcore_axis_name="s")


@functools.partial(jax.jit, static_argnums=(0, 1))
def _sc_scatter(nc, fc, hps, srcb, dstb, zeros):
    def body(*refs):
        hp_refs = refs[:nc]
        srcb_r, dstb_r, zeros_r, out_r = refs[nc:nc + 4]
        sidx, didx, vals, agg, gsem, ssem = refs[nc + 4:]
        cid = lax.axis_index("c")
        sid = lax.axis_index("s")
        wid = cid * NSUB + sid
        r0 = wid * RPW
        st = sid * STRIPE
        for c in range(nc):
            pltpu.sync_copy(zeros_r.at[pl.ds(st, STRIPE)],
                            agg.at[pl.ds(st, STRIPE)])
            plsc.subcore_barrier()

            def super_body(sk, _, c=c):
                row0 = r0 + sk * SK
                pltpu.sync_copy(srcb_r.at[pl.ds(row0, SK)], sidx)
                pltpu.sync_copy(dstb_r.at[pl.ds(row0, SK)], didx)

                def fire_gathers(s, j0):
                    for r in range(K4):
                        pltpu.async_copy(hp_refs[c].at[sidx.at[j0 + r]],
                                         vals.at[s, r], gsem)

                def drain_gathers(s):
                    for r in range(K4):
                        pltpu.make_async_copy(hp_refs[c].at[sidx.at[0]],
                                              vals.at[s, r], gsem).wait()

                def do_scatters(s, j0):
                    sds = [
                        pltpu.async_copy(vals.at[s, r],
                                         agg.at[didx.at[j0 + r]],
                                         ssem, add=True)
                        for r in range(K4)
                    ]
                    for sd in sds:
                        sd.wait()

                fire_gathers(0, 0)

                def pair_body(p, _):
                    j0 = 2 * p * K4
                    fire_gathers(1, j0 + K4)
                    drain_gathers(0)
                    do_scatters(0, j0)
                    fire_gathers(0, j0 + 2 * K4)
                    drain_gathers(1)
                    do_scatters(1, j0 + K4)
                    return 0

                lax.fori_loop(0, NPAIR - 1, pair_body, 0)
                j0 = (NG - 2) * K4
                fire_gathers(1, j0 + K4)
                drain_gathers(0)
                do_scatters(0, j0)
                drain_gathers(1)
                do_scatters(1, j0 + K4)
                return 0

            lax.fori_loop(0, NSK, super_body, 0)
            plsc.subcore_barrier()
            pltpu.sync_copy(agg.at[pl.ds(st, STRIPE)],
                            out_r.at[pl.ds(st, STRIPE),
                                     pl.ds(cid * 64 + c * fc, fc)])
            plsc.subcore_barrier()

    k = pl.kernel(
        body,
        out_type=jax.ShapeDtypeStruct((NP, 128), jnp.float32),
        mesh=_sc_mesh(),
        scratch_types=[
            pltpu.VMEM((SK, B), jnp.int32),
            pltpu.VMEM((SK, B), jnp.int32),
            pltpu.VMEM((2, K4, B, fc), jnp.float32),
            pltpu.VMEM_SHARED((NP, fc), jnp.float32),
            pltpu.SemaphoreType.DMA,
            pltpu.SemaphoreType.DMA,
        ],
        compiler_params=pltpu.CompilerParams(use_tc_tiling_on_sc=False),
    )
    return k(*hps, srcb, dstb, zeros)


@jax.jit
def _sc_degree(dstb, ones, zeros):
    def body(dstb_r, ones_r, zeros_r, out_r, didx, ones_v, agg, sem):
        cid = lax.axis_index("c")
        sid = lax.axis_index("s")
        wid = cid * NSUB + sid
        r0 = wid * RPW
        st = sid * STRIPE
        pltpu.sync_copy(ones_r, ones_v)
        pltpu.sync_copy(zeros_r.at[pl.ds(st, STRIPE)],
                        agg.at[pl.ds(st, STRIPE)])
        plsc.subcore_barrier()

        def super_body(sk, _):
            row0 = r0 + sk * DSK
            pltpu.sync_copy(dstb_r.at[pl.ds(row0, DSK)], didx)

            def group_body(g, _):
                j0 = g * DK
                sds = [
                    pltpu.async_copy(ones_v, agg.at[didx.at[j0 + r]],
                                     sem, add=True)
                    for r in range(DK)
                ]
                for sd in sds:
                    sd.wait()
                return 0

            return lax.fori_loop(0, DNG, group_body, 0)

        lax.fori_loop(0, DNSK, super_body, 0)
        plsc.subcore_barrier()
        pltpu.sync_copy(agg.at[pl.ds(st, STRIPE)],
                        out_r.at[pl.ds(st, STRIPE), pl.ds(cid * 8, 8)])

    k = pl.kernel(
        body,
        out_type=jax.ShapeDtypeStruct((NP, 128), jnp.float32),
        mesh=_sc_mesh(),
        scratch_types=[
            pltpu.VMEM((DSK, B), jnp.int32),
            pltpu.VMEM((B, 8), jnp.float32),
            pltpu.VMEM_SHARED((NP, 8), jnp.float32),
            pltpu.SemaphoreType.DMA,
        ],
        compiler_params=pltpu.CompilerParams(use_tc_tiling_on_sc=False),
    )
    return k(dstb, ones, zeros)


def _prep_body(deg_ref, out_ref):
    d = deg_ref[...]
    out_ref[...] = lax.rsqrt(d[:, 0:1] + d[:, 8:9] + 1.0)


@jax.jit
def _tc_prep(deg):
    return pl.pallas_call(
        _prep_body,
        grid=(_NGRID,),
        in_specs=[pl.BlockSpec((_NB, 128), lambda i: (i, 0))],
        out_specs=pl.BlockSpec((_NB, 1), lambda i: (i, 0)),
        out_shape=jax.ShapeDtypeStruct((N, 1), jnp.float32),
    )(deg)


def _small_matmul(x, w):
    return jnp.dot(x, w, preferred_element_type=jnp.float32,
                   precision=lax.Precision.HIGHEST)


def _mm_body(nc, fc, x_ref, w_ref, dinv_ref, *out_refs):
    h = _small_matmul(x_ref[...], w_ref[...]) * dinv_ref[...]
    for c in range(nc):
        out_refs[c][...] = h[:, c * fc:(c + 1) * fc]


@functools.partial(jax.jit, static_argnums=(0, 1))
def _tc_matmul(nc, fc, x, w, dinv):
    fin = x.shape[1]
    f = w.shape[1]
    return pl.pallas_call(
        functools.partial(_mm_body, nc, fc),
        grid=(_NGRID,),
        in_specs=[
            pl.BlockSpec((_NB, fin), lambda i: (i, 0)),
            pl.BlockSpec((fin, f), lambda i: (0, 0)),
            pl.BlockSpec((_NB, 1), lambda i: (i, 0)),
        ],
        out_specs=[pl.BlockSpec((_NB, fc), lambda i: (i, 0))] * nc,
        out_shape=[jax.ShapeDtypeStruct((N, fc), jnp.float32)] * nc,
    )(x, w, dinv)


def _fuse_body(ncl, fcl, ncn, fcn, agg_ref, *refs):
    hp_refs = refs[:ncl]
    dinv_ref, b_ref, w_ref = refs[ncl:ncl + 3]
    out_refs = refs[ncl + 3:]
    dinv = dinv_ref[...]
    b = b_ref[...]
    a = agg_ref[...]
    parts = [
        dinv * (a[:, c * fcl:(c + 1) * fcl]
                + a[:, 64 + c * fcl:64 + (c + 1) * fcl]
                + hp_refs[c][...])
        + b[:, c * fcl:(c + 1) * fcl]
        for c in range(ncl)
    ]
    x = parts[0] if ncl == 1 else jnp.concatenate(parts, axis=1)
    x = jnp.maximum(x, 0.0)
    h = _small_matmul(x, w_ref[...]) * dinv
    for c in range(ncn):
        out_refs[c][...] = h[:, c * fcn:(c + 1) * fcn]


@functools.partial(jax.jit, static_argnums=(0, 1, 2, 3))
def _tc_fuse(ncl, fcl, ncn, fcn, agg, hps, dinv, b, w):
    fl = ncl * fcl
    fn = w.shape[1]
    return pl.pallas_call(
        functools.partial(_fuse_body, ncl, fcl, ncn, fcn),
        grid=(_NGRID,),
        in_specs=[
            pl.BlockSpec((_NB, 128), lambda i: (i, 0)),
        ] + [
            pl.BlockSpec((_NB, fcl), lambda i: (i, 0)),
        ] * ncl + [
            pl.BlockSpec((_NB, 1), lambda i: (i, 0)),
            pl.BlockSpec((1, fl), lambda i: (0, 0)),
            pl.BlockSpec((fl, fn), lambda i: (0, 0)),
        ],
        out_specs=[pl.BlockSpec((_NB, fcn), lambda i: (i, 0))] * ncn,
        out_shape=[jax.ShapeDtypeStruct((N, fcn), jnp.float32)] * ncn,
    )(agg, *hps, dinv, b.reshape(1, fl), w)


def _final_body(agg_ref, hp_ref, dinv_ref, b_ref, out_ref):
    dinv = dinv_ref[...]
    a = agg_ref[...]
    t = dinv * (a[:, 0:8] + a[:, 64:72] + hp_ref[...]) + b_ref[...]
    t = jnp.maximum(t, 0.0)
    s = jnp.sum(t, axis=1, keepdims=True)
    out_ref[...] = jax.nn.sigmoid(s)


@jax.jit
def _tc_final(agg, hp, dinv, b):
    return pl.pallas_call(
        _final_body,
        grid=(_NGRID,),
        in_specs=[
            pl.BlockSpec((_NB, 128), lambda i: (i, 0)),
            pl.BlockSpec((_NB, 8), lambda i: (i, 0)),
            pl.BlockSpec((_NB, 1), lambda i: (i, 0)),
            pl.BlockSpec((1, 8), lambda i: (0, 0)),
        ],
        out_specs=pl.BlockSpec((_NB, 1), lambda i: (i, 0)),
        out_shape=jax.ShapeDtypeStruct((N, 1), jnp.float32),
    )(agg, hp, dinv, b.reshape(1, 8))


def kernel(value, edge_index, W1, b1, W2, b2, W3, b3, W4, b4, W5, b5):
    src = edge_index[0]
    dst = edge_index[1]
    npad = EP - E
    srcb = jnp.concatenate([src, jnp.zeros((npad,), jnp.int32)]).reshape(RT, B)
    dstb = jnp.concatenate([dst, jnp.full((npad,), N, jnp.int32)]).reshape(RT, B)

    ones8 = jnp.ones((B, 8), jnp.float32)
    zeros8 = jnp.zeros((NP, 8), jnp.float32)
    deg = _sc_degree(dstb, ones8, zeros8)
    dinv = _tc_prep(deg)

    W5p = jnp.concatenate([W5, jnp.zeros((W5.shape[0], 4), jnp.float32)], 1)
    b5p = jnp.concatenate([b5, jnp.zeros((4,), jnp.float32)])
    ws = [W1, W2, W3, W4, W5p]
    bs = [b1, b2, b3, b4, b5p]

    nc, fc = _chunking(_DIMS[1])
    hps = _tc_matmul(nc, fc, value, ws[0], dinv)
    for l in range(5):
        zeros = jnp.zeros((NP, fc), jnp.float32)
        agg = _sc_scatter(nc, fc, hps, srcb, dstb, zeros)
        if l < 4:
            ncn, fcn = _chunking(_DIMS[l + 2])
            hps = _tc_fuse(nc, fc, ncn, fcn, agg, hps, dinv, bs[l], ws[l + 1])
            nc, fc = ncn, fcn
        else:
            out = _tc_final(agg, hps[0], dinv, bs[l])
    return out.reshape(N)
